# Initial kernel scaffold; baseline (speedup 1.0000x reference)
import jax
import jax.numpy as jnp
from jax.experimental import pallas as pl


def kernel(feat1, feat2, reliability, aflow):
    raise NotImplementedError("write your pallas kernel here")


# speedup vs baseline: 1.9430x; 1.9430x over previous
import functools

import jax
import jax.numpy as jnp
from jax import lax
from jax.experimental import pallas as pl
from jax.experimental.pallas import tpu as pltpu
from jax.experimental.pallas import tpu_sc as plsc

_SUB = 8
_N_NEG = 512
_B_PNP = 2.0
_ALPHA = 1.0
_ANNEAL = 0.01
_MAGIC = float(1.5 * (2.0 ** 23))

_L = 16


def _sc_gather(targ, nidx, f2flat, B, C, H, W, Q):
    HW = H * W
    N = _N_NEG
    mesh = plsc.VectorSubcoreMesh(core_axis_name="c", subcore_axis_name="s")
    NC = mesh.num_cores
    NS = mesh.num_subcores
    NW = NC * NS
    WPB = NW // B
    CPW = C // WPB
    PCH = 112
    NCH = 128

    @functools.partial(
        pl.kernel,
        out_type=[
            jax.ShapeDtypeStruct((B, C, Q), jnp.float32),
            jax.ShapeDtypeStruct((B, C, N), jnp.float32),
            jax.ShapeDtypeStruct((B, Q), jnp.float32),
        ],
        mesh=mesh,
        scratch_types=[
            pltpu.VMEM((Q,), jnp.float32),
            pltpu.VMEM((Q,), jnp.float32),
            pltpu.VMEM((Q,), jnp.float32),
            pltpu.VMEM((Q,), jnp.int32),
            pltpu.VMEM((N,), jnp.int32),
            pltpu.VMEM((Q,), jnp.float32),
            pltpu.VMEM((N,), jnp.float32),
            pltpu.SemaphoreType.DMA,
        ],
    )
    def k(targ_hbm, nidx_hbm, f2_hbm, pf_hbm, nf_hbm, msk_hbm,
          txv, tyv, mskv, idxv, nidxv, gv, ngv, sem):
        wid = lax.axis_index("s") * NC + lax.axis_index("c")
        b = wid // WPB
        cblk = (wid % WPB) * CPW
        base_off = (b * C + cblk) * HW

        pltpu.sync_copy(targ_hbm.at[b, 0], txv)
        pltpu.sync_copy(targ_hbm.at[b, 1], tyv)
        pltpu.sync_copy(nidx_hbm.at[b], nidxv)

        def idx_body(i, _):
            tx = txv[pl.ds(i * _L, _L)]
            ty = tyv[pl.ds(i * _L, _L)]
            rx = (tx + _MAGIC) - _MAGIC
            ry = (ty + _MAGIC) - _MAGIC
            ok = ((rx >= 0.0) & (rx <= W - 1.0)
                  & (ry >= 0.0) & (ry <= H - 1.0))
            cx = jnp.clip(rx, 0.0, W - 1.0).astype(jnp.int32)
            cy = jnp.clip(ry, 0.0, H - 1.0).astype(jnp.int32)
            mskv[pl.ds(i * _L, _L)] = jnp.where(ok, 1.0, 0.0)
            idxv[pl.ds(i * _L, _L)] = cy * W + cx + base_off
            return 0

        lax.fori_loop(0, Q // _L, idx_body, 0, unroll=True)

        def nidx_body(i, _):
            nidxv[pl.ds(i * _L, _L)] = nidxv[pl.ds(i * _L, _L)] + base_off
            return 0

        lax.fori_loop(0, N // _L, nidx_body, 0, unroll=True)

        @pl.when(cblk == 0)
        def _():
            pltpu.sync_copy(mskv, msk_hbm.at[b])

        def c_body(ci, _):
            c = cblk + ci
            cps = []
            for j in range(Q // PCH):
                cps.append(pltpu.async_copy(
                    f2_hbm.at[idxv.at[pl.ds(j * PCH, PCH)]],
                    gv.at[pl.ds(j * PCH, PCH)], sem))
            for j in range(N // NCH):
                cps.append(pltpu.async_copy(
                    f2_hbm.at[nidxv.at[pl.ds(j * NCH, NCH)]],
                    ngv.at[pl.ds(j * NCH, NCH)], sem))
            for cp in cps:
                cp.wait()
            pltpu.sync_copy(gv, pf_hbm.at[b, c])
            pltpu.sync_copy(ngv, nf_hbm.at[b, c])

            def bump(i, _):
                idxv[pl.ds(i * _L, _L)] = idxv[pl.ds(i * _L, _L)] + HW
                return 0

            lax.fori_loop(0, Q // _L, bump, 0, unroll=True)

            def nbump(i, _):
                nidxv[pl.ds(i * _L, _L)] = nidxv[pl.ds(i * _L, _L)] + HW
                return 0

            lax.fori_loop(0, N // _L, nbump, 0, unroll=True)
            return 0

        lax.fori_loop(0, CPW, c_body, 0)

    return k(targ, nidx, f2flat)


def _tc_loss(qfeat, pfeat, nfeat, qconf, msk):
    B, C, Q = qfeat.shape
    N = nfeat.shape[2]

    def body(q_ref, p_ref, n_ref, qc_ref, m_ref, out_ref, acc_ref):
        bidx = pl.program_id(0)

        @pl.when(bidx == 0)
        def _():
            acc_ref[0] = 0.0
            acc_ref[1] = 0.0

        q = q_ref[0]
        p = p_ref[0]
        n = n_ref[0]
        qn = q / (jnp.sqrt(jnp.sum(q * q, axis=0, keepdims=True)) + 1e-8)
        pn = p / (jnp.sqrt(jnp.sum(p * p, axis=0, keepdims=True)) + 1e-8)
        nn = n / (jnp.sqrt(jnp.sum(n * n, axis=0, keepdims=True)) + 1e-8)
        pos = jnp.sum(qn * pn, axis=0, keepdims=True)
        negT = lax.dot_general(nn, qn, (((0,), (0,)), ((), ())),
                               preferred_element_type=jnp.float32)
        d = jnp.sum(jax.nn.sigmoid((negT - pos) * (1.0 / _ANNEAL)),
                    axis=0, keepdims=True)
        base = 1.0 + _ALPHA * d
        pnp = 1.0 - 1.0 / (base * base)
        qc = qc_ref[0]
        m = m_ref[0]
        pix = pnp * qc + 0.5 * (1.0 - qc)
        acc_ref[0] += jnp.sum(pix * m)
        acc_ref[1] += jnp.sum(m)

        @pl.when(bidx == B - 1)
        def _():
            out_ref[...] = jnp.full((1, 1), acc_ref[0] / (acc_ref[1] + 1e-8),
                                    dtype=jnp.float32)

    out = pl.pallas_call(
        body,
        grid=(B,),
        in_specs=[
            pl.BlockSpec((1, C, Q), lambda b: (b, 0, 0)),
            pl.BlockSpec((1, C, Q), lambda b: (b, 0, 0)),
            pl.BlockSpec((1, C, N), lambda b: (b, 0, 0)),
            pl.BlockSpec((1, 1, Q), lambda b: (b, 0, 0)),
            pl.BlockSpec((1, 1, Q), lambda b: (b, 0, 0)),
        ],
        out_specs=pl.BlockSpec((1, 1), lambda b: (0, 0)),
        out_shape=jax.ShapeDtypeStruct((1, 1), jnp.float32),
        scratch_shapes=[pltpu.SMEM((2,), jnp.float32)],
    )(qfeat, pfeat, nfeat, qconf.reshape(B, 1, Q), msk.reshape(B, 1, Q))
    return out[0, 0]


def kernel(feat1, feat2, reliability, aflow):
    B, C, H, W = feat1.shape
    h = _SUB // 2
    Q = (H // _SUB) * (W // _SUB)

    qfeat = feat1[:, :, h::_SUB, h::_SUB].reshape(B, C, Q)
    qconf = reliability[:, 0, h::_SUB, h::_SUB].reshape(B, Q)
    targ = aflow[:, :, h::_SUB, h::_SUB].reshape(B, 2, Q)

    nidx = jax.random.randint(jax.random.key(42), (B, _N_NEG), 0, H * W)
    nidx = nidx.astype(jnp.int32)

    f2flat = feat2.reshape(B * C * H * W)
    pfeat, nfeat, msk = _sc_gather(targ, nidx, f2flat, B, C, H, W, Q)
    return _tc_loss(qfeat, pfeat, nfeat, qconf, msk)
